# baseline (speedup 1.0000x reference)
import functools

import jax
import jax.numpy as jnp
from jax import lax
from jax.experimental import pallas as pl
from jax.experimental.pallas import tpu as pltpu
from jax.experimental.pallas import tpu_sc as plsc

N = 10000
D = 128
E = 320000
NC = 2
NS = 16
NW = NC * NS
EPT = E // NW
C = 80
NCHUNK = EPT // C
RPT = N // NS
DW = 128

@functools.lru_cache(maxsize=None)
def _sc_kernels():
    mesh = plsc.VectorSubcoreMesh(
        core_axis_name="c", subcore_axis_name="s",
        num_cores=NC, num_subcores=NS)

    @functools.partial(
        pl.kernel,
        out_type=jax.ShapeDtypeStruct((NC, N, DW), jnp.float32),
        mesh=mesh,
        scratch_types=[
            pltpu.VMEM((NCHUNK, C), jnp.int32),
            pltpu.VMEM((C, DW), jnp.float32),
            pltpu.VMEM_SHARED((N, DW), jnp.float32),
        ],
    )
    def deg_kernel(dst_hbm, ones_hbm, zeros_hbm, out_hbm, dst_v, ones_v, acc):
        cid = lax.axis_index("c")
        sid = lax.axis_index("s")
        wid = cid * NS + sid
        pltpu.sync_copy(dst_hbm.at[wid], dst_v)
        pltpu.sync_copy(ones_hbm, ones_v)

        @pl.when(sid == 0)
        def _():
            pltpu.sync_copy(zeros_hbm, acc)

        plsc.subcore_barrier()

        @pl.loop(0, NCHUNK)
        def _(i):
            pltpu.sync_copy(ones_v, acc.at[dst_v.at[i]], add=True)

        plsc.subcore_barrier()

        @pl.when(sid == 0)
        def _():
            pltpu.sync_copy(acc, out_hbm.at[cid])

    @functools.partial(
        pl.kernel,
        out_type=jax.ShapeDtypeStruct((NC, N, D), jnp.float32),
        mesh=mesh,
        scratch_types=[
            pltpu.VMEM((EPT,), jnp.int32),
            pltpu.VMEM((NCHUNK, C), jnp.int32),
            pltpu.VMEM((C, D), jnp.float32),
            pltpu.VMEM((C, D), jnp.float32),
            pltpu.VMEM_SHARED((N, D), jnp.float32),
            pltpu.SemaphoreType.DMA,
            pltpu.SemaphoreType.DMA,
            pltpu.SemaphoreType.DMA,
            pltpu.SemaphoreType.DMA,
        ],
    )
    def agg_kernel(src_hbm, dst_hbm, ys_hbm, zeros_hbm, out_hbm,
                   src_v, dst_v, buf0, buf1, acc, sem0, sem1, ssem0, ssem1):
        cid = lax.axis_index("c")
        sid = lax.axis_index("s")
        wid = cid * NS + sid
        pltpu.sync_copy(src_hbm.at[wid], src_v)
        pltpu.sync_copy(dst_hbm.at[wid], dst_v)

        @pl.when(sid == 0)
        def _():
            pltpu.sync_copy(zeros_hbm, acc)

        plsc.subcore_barrier()

        def gather(i, buf, sem):
            pltpu.async_copy(ys_hbm.at[src_v.at[pl.ds(i * C, C)]], buf, sem)

        def gwait(buf, sem):
            pltpu.make_async_copy(ys_hbm.at[src_v.at[pl.ds(0, C)]],
                                  buf, sem).wait()

        def scatter(i, buf, sem):
            return pltpu.async_copy(buf, acc.at[dst_v.at[i]], sem, add=True)

        gather(0, buf0, sem0)
        gather(1, buf1, sem1)

        @pl.loop(0, NCHUNK - 3, step=2)
        def _(i):
            gwait(buf0, sem0)
            d0 = scatter(i, buf0, ssem0)
            gwait(buf1, sem1)
            d1 = scatter(i + 1, buf1, ssem1)
            d0.wait()
            gather(i + 2, buf0, sem0)
            d1.wait()
            gather(i + 3, buf1, sem1)

        gwait(buf0, sem0)
        d0 = scatter(NCHUNK - 3, buf0, ssem0)
        gwait(buf1, sem1)
        d1 = scatter(NCHUNK - 2, buf1, ssem1)
        d0.wait()
        gather(NCHUNK - 1, buf0, sem0)
        d1.wait()
        gwait(buf0, sem0)
        pltpu.sync_copy(buf0, acc.at[dst_v.at[NCHUNK - 1]], add=True)

        plsc.subcore_barrier()

        @pl.when(sid == 0)
        def _():
            pltpu.sync_copy(acc, out_hbm.at[cid])

    return deg_kernel, agg_kernel


BLK = 400
GRID = N // BLK


def _deg_dis(degp):
    deg = degp[0, :, 0] + degp[1, :, 0] + 1.0
    dis = lax.rsqrt(deg)
    return deg, dis


def _tc1_body(degp_ref, x_ref, w_ref, b_ref, ys_ref, t_ref):
    deg, dis = _deg_dis(degp_ref[...])
    xw = jnp.dot(x_ref[...], w_ref[...], preferred_element_type=jnp.float32)
    ys_ref[...] = xw * dis[:, None]
    t_ref[...] = xw * (1.0 / deg)[:, None] + b_ref[...]


def _tc2_body(aggp_ref, degp_ref, t_ref, w_ref, b_ref, ys_ref, t2_ref):
    deg, dis = _deg_dis(degp_ref[...])
    a = aggp_ref[...]
    h = jax.nn.relu((a[0] + a[1]) * dis[:, None] + t_ref[...])
    xw = jnp.dot(h, w_ref[...], preferred_element_type=jnp.float32)
    ys_ref[...] = xw * dis[:, None]
    t2_ref[...] = xw * (1.0 / deg)[:, None] + b_ref[...]


def _tc3_body(aggp_ref, degp_ref, t_ref, act_ref, fa_ref, fb_ref, f1b_ref,
              f2w_ref, f2b_ref, q_ref):
    _, dis = _deg_dis(degp_ref[...])
    a = aggp_ref[...]
    h2 = jax.nn.relu((a[0] + a[1]) * dis[:, None] + t_ref[...])
    hc = (jnp.dot(h2, fa_ref[...], preferred_element_type=jnp.float32)
          + jnp.dot(act_ref[...], fb_ref[...],
                    preferred_element_type=jnp.float32)
          + f1b_ref[...])
    h3 = jax.nn.relu(hc)
    q_ref[...] = (jnp.sum(h3 * f2w_ref[...], axis=1) + f2b_ref[0, 0])[:, None]


def _degp_spec():
    return pl.BlockSpec((NC, BLK, DW), lambda i: (0, i, 0))


def _row_spec(w=D):
    return pl.BlockSpec((BLK, w), lambda i: (i, 0))


def _full_spec(r, c):
    return pl.BlockSpec((r, c), lambda i: (0, 0))


_tc1 = pl.pallas_call(
    _tc1_body,
    grid=(GRID,),
    in_specs=[_degp_spec(), _row_spec(), _full_spec(D, D), _full_spec(1, D)],
    out_specs=[_row_spec(), _row_spec()],
    out_shape=[jax.ShapeDtypeStruct((N, D), jnp.float32)] * 2,
)

_tc2 = pl.pallas_call(
    _tc2_body,
    grid=(GRID,),
    in_specs=[
        pl.BlockSpec((NC, BLK, D), lambda i: (0, i, 0)),
        _degp_spec(), _row_spec(), _full_spec(D, D), _full_spec(1, D),
    ],
    out_specs=[_row_spec(), _row_spec()],
    out_shape=[jax.ShapeDtypeStruct((N, D), jnp.float32)] * 2,
)

_tc3 = pl.pallas_call(
    _tc3_body,
    grid=(GRID,),
    in_specs=[
        pl.BlockSpec((NC, BLK, D), lambda i: (0, i, 0)),
        _degp_spec(), _row_spec(), _row_spec(16),
        _full_spec(D, D), _full_spec(16, D), _full_spec(1, D),
        _full_spec(1, D), _full_spec(1, 1),
    ],
    out_specs=pl.BlockSpec((BLK, 1), lambda i: (i, 0)),
    out_shape=jax.ShapeDtypeStruct((N, 1), jnp.float32),
)


def kernel(x, edge_index, action, W1, b1, W2, b2, fc1_W, fc1_b, fc2_W, fc2_b):
    src = edge_index[0].reshape(NW, EPT)
    dst = edge_index[1].reshape(NW, NCHUNK, C)
    ones8 = jnp.ones((C, DW), jnp.float32)
    zeros8 = jnp.zeros((N, DW), jnp.float32)
    zerosD = jnp.zeros((N, D), jnp.float32)

    deg_kernel, agg_kernel = _sc_kernels()
    degp = deg_kernel(dst, ones8, zeros8)
    ys1, t1 = _tc1(degp, x, W1, b1.reshape(1, D))
    aggp1 = agg_kernel(src, dst, ys1, zerosD)
    ys2, t2 = _tc2(aggp1, degp, t1, W2, b2.reshape(1, D))
    aggp2 = agg_kernel(src, dst, ys2, zerosD)
    q = _tc3(aggp2, degp, t2, action, fc1_W[:D], fc1_W[D:],
             fc1_b.reshape(1, D), fc2_W.reshape(1, D), fc2_b.reshape(1, 1))
    return q.reshape(N)


# speedup vs baseline: 1.1697x; 1.1697x over previous
import functools

import jax
import jax.numpy as jnp
from jax import lax
from jax.experimental import pallas as pl
from jax.experimental.pallas import tpu as pltpu
from jax.experimental.pallas import tpu_sc as plsc

N = 10000
D = 128
E = 320000
NC = 2
NS = 16
NW = NC * NS
EPT = E // NW
C = 80
NCHUNK = EPT // C
RPT = N // NS
DW = 128

RA = 624
RB = N - (NS - 1) * RA


def _par_copy(sid, src_ref, dst_ref):
    @pl.when(sid < NS - 1)
    def _():
        pltpu.sync_copy(src_ref.at[pl.ds(sid * RA, RA)],
                        dst_ref.at[pl.ds(sid * RA, RA)])

    @pl.when(sid == NS - 1)
    def _():
        pltpu.sync_copy(src_ref.at[pl.ds((NS - 1) * RA, RB)],
                        dst_ref.at[pl.ds((NS - 1) * RA, RB)])


@functools.lru_cache(maxsize=None)
def _sc_kernels():
    mesh = plsc.VectorSubcoreMesh(
        core_axis_name="c", subcore_axis_name="s",
        num_cores=NC, num_subcores=NS)

    @functools.partial(
        pl.kernel,
        out_type=jax.ShapeDtypeStruct((NC, N, DW), jnp.float32),
        mesh=mesh,
        scratch_types=[
            pltpu.VMEM((NCHUNK, C), jnp.int32),
            pltpu.VMEM((C, DW), jnp.float32),
            pltpu.VMEM_SHARED((N, DW), jnp.float32),
        ],
    )
    def deg_kernel(dst_hbm, ones_hbm, zeros_hbm, out_hbm, dst_v, ones_v, acc):
        cid = lax.axis_index("c")
        sid = lax.axis_index("s")
        wid = cid * NS + sid
        pltpu.sync_copy(dst_hbm.at[wid], dst_v)
        pltpu.sync_copy(ones_hbm, ones_v)
        _par_copy(sid, zeros_hbm, acc)
        plsc.subcore_barrier()

        @pl.loop(0, NCHUNK)
        def _(i):
            pltpu.sync_copy(ones_v, acc.at[dst_v.at[i]], add=True)

        plsc.subcore_barrier()
        _par_copy(sid, acc, out_hbm.at[cid])

    @functools.partial(
        pl.kernel,
        out_type=jax.ShapeDtypeStruct((NC, N, D), jnp.float32),
        mesh=mesh,
        scratch_types=[
            pltpu.VMEM((EPT,), jnp.int32),
            pltpu.VMEM((NCHUNK, C), jnp.int32),
            pltpu.VMEM((C, D), jnp.float32),
            pltpu.VMEM((C, D), jnp.float32),
            pltpu.VMEM_SHARED((N, D), jnp.float32),
            pltpu.SemaphoreType.DMA,
            pltpu.SemaphoreType.DMA,
        ],
    )
    def agg_kernel(src_hbm, dst_hbm, ys_hbm, zeros_hbm, out_hbm,
                   src_v, dst_v, buf0, buf1, acc, sem0, sem1):
        cid = lax.axis_index("c")
        sid = lax.axis_index("s")
        wid = cid * NS + sid
        pltpu.sync_copy(src_hbm.at[wid], src_v)
        pltpu.sync_copy(dst_hbm.at[wid], dst_v)
        _par_copy(sid, zeros_hbm, acc)
        plsc.subcore_barrier()

        def gather(i, buf, sem):
            pltpu.async_copy(ys_hbm.at[src_v.at[pl.ds(i * C, C)]], buf, sem)

        def gwait(buf, sem):
            pltpu.make_async_copy(ys_hbm.at[src_v.at[pl.ds(0, C)]],
                                  buf, sem).wait()

        gather(0, buf0, sem0)

        @pl.loop(0, NCHUNK - 1, step=2)
        def _(i):
            gather(i + 1, buf1, sem1)
            gwait(buf0, sem0)
            pltpu.sync_copy(buf0, acc.at[dst_v.at[i]], add=True)
            gather(i + 2, buf0, sem0)
            gwait(buf1, sem1)
            pltpu.sync_copy(buf1, acc.at[dst_v.at[i + 1]], add=True)

        gwait(buf0, sem0)
        pltpu.sync_copy(buf0, acc.at[dst_v.at[NCHUNK - 1]], add=True)

        plsc.subcore_barrier()
        _par_copy(sid, acc, out_hbm.at[cid])

    return deg_kernel, agg_kernel


BLK = 400
GRID = N // BLK


def _deg_dis(degp):
    deg = degp[0, :, 0] + degp[1, :, 0] + 1.0
    dis = lax.rsqrt(deg)
    return deg, dis


def _tc1_body(degp_ref, x_ref, w_ref, b_ref, ys_ref, t_ref):
    deg, dis = _deg_dis(degp_ref[...])
    xw = jnp.dot(x_ref[...], w_ref[...], preferred_element_type=jnp.float32)
    ys_ref[...] = xw * dis[:, None]
    t_ref[...] = xw * (1.0 / deg)[:, None] + b_ref[...]


def _tc2_body(aggp_ref, degp_ref, t_ref, w_ref, b_ref, ys_ref, t2_ref):
    deg, dis = _deg_dis(degp_ref[...])
    a = aggp_ref[...]
    h = jax.nn.relu((a[0] + a[1]) * dis[:, None] + t_ref[...])
    xw = jnp.dot(h, w_ref[...], preferred_element_type=jnp.float32)
    ys_ref[...] = xw * dis[:, None]
    t2_ref[...] = xw * (1.0 / deg)[:, None] + b_ref[...]


def _tc3_body(aggp_ref, degp_ref, t_ref, act_ref, fa_ref, fb_ref, f1b_ref,
              f2w_ref, f2b_ref, q_ref):
    _, dis = _deg_dis(degp_ref[...])
    a = aggp_ref[...]
    h2 = jax.nn.relu((a[0] + a[1]) * dis[:, None] + t_ref[...])
    hc = (jnp.dot(h2, fa_ref[...], preferred_element_type=jnp.float32)
          + jnp.dot(act_ref[...], fb_ref[...],
                    preferred_element_type=jnp.float32)
          + f1b_ref[...])
    h3 = jax.nn.relu(hc)
    q_ref[...] = (jnp.sum(h3 * f2w_ref[...], axis=1) + f2b_ref[0, 0])[:, None]


def _degp_spec():
    return pl.BlockSpec((NC, BLK, DW), lambda i: (0, i, 0))


def _row_spec(w=D):
    return pl.BlockSpec((BLK, w), lambda i: (i, 0))


def _full_spec(r, c):
    return pl.BlockSpec((r, c), lambda i: (0, 0))


_tc1 = pl.pallas_call(
    _tc1_body,
    grid=(GRID,),
    in_specs=[_degp_spec(), _row_spec(), _full_spec(D, D), _full_spec(1, D)],
    out_specs=[_row_spec(), _row_spec()],
    out_shape=[jax.ShapeDtypeStruct((N, D), jnp.float32)] * 2,
)

_tc2 = pl.pallas_call(
    _tc2_body,
    grid=(GRID,),
    in_specs=[
        pl.BlockSpec((NC, BLK, D), lambda i: (0, i, 0)),
        _degp_spec(), _row_spec(), _full_spec(D, D), _full_spec(1, D),
    ],
    out_specs=[_row_spec(), _row_spec()],
    out_shape=[jax.ShapeDtypeStruct((N, D), jnp.float32)] * 2,
)

_tc3 = pl.pallas_call(
    _tc3_body,
    grid=(GRID,),
    in_specs=[
        pl.BlockSpec((NC, BLK, D), lambda i: (0, i, 0)),
        _degp_spec(), _row_spec(), _row_spec(16),
        _full_spec(D, D), _full_spec(16, D), _full_spec(1, D),
        _full_spec(1, D), _full_spec(1, 1),
    ],
    out_specs=pl.BlockSpec((BLK, 1), lambda i: (i, 0)),
    out_shape=jax.ShapeDtypeStruct((N, 1), jnp.float32),
)


def kernel(x, edge_index, action, W1, b1, W2, b2, fc1_W, fc1_b, fc2_W, fc2_b):
    src = edge_index[0].reshape(NW, EPT)
    dst = edge_index[1].reshape(NW, NCHUNK, C)
    ones8 = jnp.ones((C, DW), jnp.float32)
    zeros8 = jnp.zeros((N, DW), jnp.float32)
    zerosD = jnp.zeros((N, D), jnp.float32)

    deg_kernel, agg_kernel = _sc_kernels()
    degp = deg_kernel(dst, ones8, zeros8)
    ys1, t1 = _tc1(degp, x, W1, b1.reshape(1, D))
    aggp1 = agg_kernel(src, dst, ys1, zerosD)
    ys2, t2 = _tc2(aggp1, degp, t1, W2, b2.reshape(1, D))
    aggp2 = agg_kernel(src, dst, ys2, zerosD)
    q = _tc3(aggp2, degp, t2, action, fc1_W[:D], fc1_W[D:],
             fc1_b.reshape(1, D), fc2_W.reshape(1, D), fc2_b.reshape(1, 1))
    return q.reshape(N)


# speedup vs baseline: 1.1727x; 1.0026x over previous
import functools

import jax
import jax.numpy as jnp
from jax import lax
from jax.experimental import pallas as pl
from jax.experimental.pallas import tpu as pltpu
from jax.experimental.pallas import tpu_sc as plsc

N = 10000
D = 128
E = 320000
NC = 2
NS = 16
NW = NC * NS
EPT = E // NW
C = 80
NCHUNK = EPT // C
RPT = N // NS
DW = 128

RA = 624
RB = N - (NS - 1) * RA


def _par_copy(sid, src_ref, dst_ref):
    @pl.when(sid < NS - 1)
    def _():
        pltpu.sync_copy(src_ref.at[pl.ds(sid * RA, RA)],
                        dst_ref.at[pl.ds(sid * RA, RA)])

    @pl.when(sid == NS - 1)
    def _():
        pltpu.sync_copy(src_ref.at[pl.ds((NS - 1) * RA, RB)],
                        dst_ref.at[pl.ds((NS - 1) * RA, RB)])


@functools.lru_cache(maxsize=None)
def _sc_kernels():
    mesh = plsc.VectorSubcoreMesh(
        core_axis_name="c", subcore_axis_name="s",
        num_cores=NC, num_subcores=NS)

    @functools.partial(
        pl.kernel,
        out_type=jax.ShapeDtypeStruct((NC, N, DW), jnp.float32),
        mesh=mesh,
        scratch_types=[
            pltpu.VMEM((NCHUNK, C), jnp.int32),
            pltpu.VMEM((C, DW), jnp.float32),
            pltpu.VMEM_SHARED((N, DW), jnp.float32),
        ],
    )
    def deg_kernel(dst_hbm, ones_hbm, zeros_hbm, out_hbm, dst_v, ones_v, acc):
        cid = lax.axis_index("c")
        sid = lax.axis_index("s")
        wid = cid * NS + sid
        pltpu.sync_copy(dst_hbm.at[wid], dst_v)
        pltpu.sync_copy(ones_hbm, ones_v)
        _par_copy(sid, zeros_hbm, acc)
        plsc.subcore_barrier()

        @pl.loop(0, NCHUNK)
        def _(i):
            pltpu.sync_copy(ones_v, acc.at[dst_v.at[i]], add=True)

        plsc.subcore_barrier()
        _par_copy(sid, acc, out_hbm.at[cid])

    @functools.partial(
        pl.kernel,
        out_type=jax.ShapeDtypeStruct((NC, N, D), jnp.float32),
        mesh=mesh,
        scratch_types=[
            pltpu.VMEM((EPT,), jnp.int32),
            pltpu.VMEM((NCHUNK, C), jnp.int32),
            pltpu.VMEM((C, D), jnp.float32),
            pltpu.VMEM((C, D), jnp.float32),
            pltpu.VMEM_SHARED((N, D), jnp.float32),
            pltpu.SemaphoreType.DMA,
            pltpu.SemaphoreType.DMA,
        ],
    )
    def agg_kernel(src_hbm, dst_hbm, ys_hbm, zeros_hbm, out_hbm,
                   src_v, dst_v, buf0, buf1, acc, sem0, sem1):
        cid = lax.axis_index("c")
        sid = lax.axis_index("s")
        wid = cid * NS + sid
        pltpu.sync_copy(src_hbm.at[wid], src_v)
        pltpu.sync_copy(dst_hbm.at[wid], dst_v)
        _par_copy(sid, zeros_hbm, acc)
        plsc.subcore_barrier()

        def gather(i, buf, sem):
            pltpu.async_copy(ys_hbm.at[src_v.at[pl.ds(i * C, C)]], buf, sem)

        def gwait(buf, sem):
            pltpu.make_async_copy(ys_hbm.at[src_v.at[pl.ds(0, C)]],
                                  buf, sem).wait()

        gather(0, buf0, sem0)

        @pl.loop(0, NCHUNK - 1, step=2)
        def _(i):
            gather(i + 1, buf1, sem1)
            gwait(buf0, sem0)
            pltpu.sync_copy(buf0, acc.at[dst_v.at[i]], add=True)
            gather(i + 2, buf0, sem0)
            gwait(buf1, sem1)
            pltpu.sync_copy(buf1, acc.at[dst_v.at[i + 1]], add=True)

        gwait(buf0, sem0)
        pltpu.sync_copy(buf0, acc.at[dst_v.at[NCHUNK - 1]], add=True)

        plsc.subcore_barrier()
        _par_copy(sid, acc, out_hbm.at[cid])

    return deg_kernel, agg_kernel


BLK = 400
GRID = N // BLK


def _deg_dis(degp):
    deg = degp[0, :, 0] + degp[1, :, 0] + 1.0
    dis = lax.rsqrt(deg)
    return deg, dis


def _tcmm_body(x_ref, w_ref, xw_ref):
    xw_ref[...] = jnp.dot(x_ref[...], w_ref[...],
                          preferred_element_type=jnp.float32)


def _tc1_body(degp_ref, xw_ref, b_ref, ys_ref, t_ref):
    deg, dis = _deg_dis(degp_ref[...])
    xw = xw_ref[...]
    ys_ref[...] = xw * dis[:, None]
    t_ref[...] = xw * (1.0 / deg)[:, None] + b_ref[...]


def _tc2_body(aggp_ref, degp_ref, t_ref, w_ref, b_ref, ys_ref, t2_ref):
    deg, dis = _deg_dis(degp_ref[...])
    a = aggp_ref[...]
    h = jax.nn.relu((a[0] + a[1]) * dis[:, None] + t_ref[...])
    xw = jnp.dot(h, w_ref[...], preferred_element_type=jnp.float32)
    ys_ref[...] = xw * dis[:, None]
    t2_ref[...] = xw * (1.0 / deg)[:, None] + b_ref[...]


def _tc3_body(aggp_ref, degp_ref, t_ref, act_ref, fa_ref, fb_ref, f1b_ref,
              f2w_ref, f2b_ref, q_ref):
    _, dis = _deg_dis(degp_ref[...])
    a = aggp_ref[...]
    h2 = jax.nn.relu((a[0] + a[1]) * dis[:, None] + t_ref[...])
    hc = (jnp.dot(h2, fa_ref[...], preferred_element_type=jnp.float32)
          + jnp.dot(act_ref[...], fb_ref[...],
                    preferred_element_type=jnp.float32)
          + f1b_ref[...])
    h3 = jax.nn.relu(hc)
    q_ref[...] = (jnp.sum(h3 * f2w_ref[...], axis=1) + f2b_ref[0, 0])[:, None]


def _degp_spec():
    return pl.BlockSpec((NC, BLK, DW), lambda i: (0, i, 0))


def _row_spec(w=D):
    return pl.BlockSpec((BLK, w), lambda i: (i, 0))


def _full_spec(r, c):
    return pl.BlockSpec((r, c), lambda i: (0, 0))


_tcmm = pl.pallas_call(
    _tcmm_body,
    grid=(GRID,),
    in_specs=[_row_spec(), _full_spec(D, D)],
    out_specs=_row_spec(),
    out_shape=jax.ShapeDtypeStruct((N, D), jnp.float32),
)

_tc1 = pl.pallas_call(
    _tc1_body,
    grid=(GRID,),
    in_specs=[_degp_spec(), _row_spec(), _full_spec(1, D)],
    out_specs=[_row_spec(), _row_spec()],
    out_shape=[jax.ShapeDtypeStruct((N, D), jnp.float32)] * 2,
)

_tc2 = pl.pallas_call(
    _tc2_body,
    grid=(GRID,),
    in_specs=[
        pl.BlockSpec((NC, BLK, D), lambda i: (0, i, 0)),
        _degp_spec(), _row_spec(), _full_spec(D, D), _full_spec(1, D),
    ],
    out_specs=[_row_spec(), _row_spec()],
    out_shape=[jax.ShapeDtypeStruct((N, D), jnp.float32)] * 2,
)

_tc3 = pl.pallas_call(
    _tc3_body,
    grid=(GRID,),
    in_specs=[
        pl.BlockSpec((NC, BLK, D), lambda i: (0, i, 0)),
        _degp_spec(), _row_spec(), _row_spec(16),
        _full_spec(D, D), _full_spec(16, D), _full_spec(1, D),
        _full_spec(1, D), _full_spec(1, 1),
    ],
    out_specs=pl.BlockSpec((BLK, 1), lambda i: (i, 0)),
    out_shape=jax.ShapeDtypeStruct((N, 1), jnp.float32),
)


def kernel(x, edge_index, action, W1, b1, W2, b2, fc1_W, fc1_b, fc2_W, fc2_b):
    src = edge_index[0].reshape(NW, EPT)
    dst = edge_index[1].reshape(NW, NCHUNK, C)
    ones8 = jnp.ones((C, DW), jnp.float32)
    zeros8 = jnp.zeros((N, DW), jnp.float32)
    zerosD = jnp.zeros((N, D), jnp.float32)

    deg_kernel, agg_kernel = _sc_kernels()
    degp = deg_kernel(dst, ones8, zeros8)
    xw1 = _tcmm(x, W1)
    ys1, t1 = _tc1(degp, xw1, b1.reshape(1, D))
    aggp1 = agg_kernel(src, dst, ys1, zerosD)
    ys2, t2 = _tc2(aggp1, degp, t1, W2, b2.reshape(1, D))
    aggp2 = agg_kernel(src, dst, ys2, zerosD)
    q = _tc3(aggp2, degp, t2, action, fc1_W[:D], fc1_W[D:],
             fc1_b.reshape(1, D), fc2_W.reshape(1, D), fc2_b.reshape(1, 1))
    return q.reshape(N)
